# Initial kernel scaffold; baseline (speedup 1.0000x reference)
import jax
import jax.numpy as jnp
from jax.experimental import pallas as pl


def kernel(x, edge_index, batch, W_l1, W_r1, b1, W_l2, W_r2, b2, bn_w, bn_b, fc_w, fc_b):
    raise NotImplementedError("write your pallas kernel here")


# speedup vs baseline: 3.4497x; 3.4497x over previous
import jax
import jax.numpy as jnp
from jax import lax
from jax.experimental import pallas as pl
from jax.experimental.pallas import tpu as pltpu
from jax.experimental.pallas import tpu_sc as plsc

NC = 2
NS = 16
CW = 8
CHUNK = 80
EPS = 1e-5
G = 64
_PREC = lax.Precision.HIGHEST


def _sc_agg(table, src, dst, ones, zf, zcf, with_counts):
    n2, dh = table.shape
    n = n2 // NC
    e = src.shape[0]
    ept = e // NS
    n_chunks = ept // CHUNK
    assert ept % CHUNK == 0
    rpt = (n // NS) // 8 * 8
    tail = n - NS * rpt
    assert tail % 8 == 0 and tail <= rpt

    out_type = [jax.ShapeDtypeStruct((n2, dh), jnp.float32)]
    scratch = [
        pltpu.VMEM((CHUNK,), jnp.int32),
        pltpu.VMEM((CHUNK,), jnp.int32),
        pltpu.VMEM((CHUNK, dh), jnp.float32),
        pltpu.VMEM((rpt, dh), jnp.float32),
        pltpu.VMEM_SHARED((n, dh), jnp.float32),
        pltpu.SemaphoreType.DMA,
    ]
    if with_counts:
        out_type.append(jax.ShapeDtypeStruct((n, CW), jnp.float32))
        scratch += [
            pltpu.VMEM((CHUNK, CW), jnp.float32),
            pltpu.VMEM((rpt, CW), jnp.float32),
            pltpu.VMEM_SHARED((n, CW), jnp.float32),
        ]

    mesh = plsc.VectorSubcoreMesh(core_axis_name="c", subcore_axis_name="s",
                                  num_cores=NC, num_subcores=NS)

    def body(*refs):
        if with_counts:
            (table_h, src_h, dst_h, ones_h, zf_h, zcf_h, acc_o, cnt_o,
             idx_v, dst_v, rows_v, zbuf, acc_sh, sem,
             ones_v, cbuf, cnt_sh) = refs
        else:
            (table_h, src_h, dst_h, ones_h, zf_h, zcf_h, acc_o,
             idx_v, dst_v, rows_v, zbuf, acc_sh, sem) = refs
        c = lax.axis_index("c")
        s = lax.axis_index("s")
        r0 = s * rpt

        pltpu.sync_copy(zf_h, zbuf)
        pltpu.sync_copy(zbuf, acc_sh.at[pl.ds(r0, rpt)])
        if with_counts:
            pltpu.sync_copy(ones_h, ones_v)
            pltpu.sync_copy(zcf_h, cbuf)

            @pl.when(c == 0)
            def _():
                pltpu.sync_copy(cbuf, cnt_sh.at[pl.ds(r0, rpt)])
        if tail:
            @pl.when(s == NS - 1)
            def _():
                pltpu.sync_copy(zbuf.at[pl.ds(0, tail)],
                                acc_sh.at[pl.ds(NS * rpt, tail)])
                if with_counts:
                    @pl.when(c == 0)
                    def _():
                        pltpu.sync_copy(cbuf.at[pl.ds(0, tail)],
                                        cnt_sh.at[pl.ds(NS * rpt, tail)])
        plsc.subcore_barrier()

        row_off = c * n

        def step(i, carry):
            base = s * ept + i * CHUNK
            pltpu.sync_copy(src_h.at[pl.ds(base, CHUNK)], idx_v)
            pltpu.sync_copy(dst_h.at[pl.ds(base, CHUNK)], dst_v)
            for k in range(CHUNK // 16):
                sl = pl.ds(k * 16, 16)
                idx_v[sl] = idx_v[sl] + row_off
            pltpu.async_copy(table_h.at[idx_v], rows_v, sem).wait()
            pltpu.sync_copy(rows_v, acc_sh.at[dst_v], add=True)
            if with_counts:
                @pl.when(c == 0)
                def _():
                    pltpu.sync_copy(ones_v, cnt_sh.at[dst_v], add=True)
            return carry
        lax.fori_loop(0, n_chunks, step, 0)
        plsc.subcore_barrier()

        pltpu.sync_copy(acc_sh.at[pl.ds(r0, rpt)], zbuf)
        pltpu.sync_copy(zbuf, acc_o.at[pl.ds(row_off + r0, rpt)])
        if with_counts:
            @pl.when(c == 0)
            def _():
                pltpu.sync_copy(cnt_sh.at[pl.ds(r0, rpt)], cbuf)
                pltpu.sync_copy(cbuf, cnt_o.at[pl.ds(r0, rpt)])
        if tail:
            @pl.when(s == NS - 1)
            def _():
                pltpu.sync_copy(acc_sh.at[pl.ds(NS * rpt, tail)],
                                zbuf.at[pl.ds(0, tail)])
                pltpu.sync_copy(zbuf.at[pl.ds(0, tail)],
                                acc_o.at[pl.ds(row_off + NS * rpt, tail)])
                if with_counts:
                    @pl.when(c == 0)
                    def _():
                        pltpu.sync_copy(cnt_sh.at[pl.ds(NS * rpt, tail)],
                                        cbuf.at[pl.ds(0, tail)])
                        pltpu.sync_copy(cbuf.at[pl.ds(0, tail)],
                                        cnt_o.at[pl.ds(NS * rpt, tail)])

    f = pl.kernel(body, out_type=tuple(out_type), mesh=mesh,
                  scratch_types=tuple(scratch),
                  compiler_params=pltpu.CompilerParams(
                      use_tc_tiling_on_sc=False))
    outs = f(table, src, dst, ones, zf, zcf)
    if with_counts:
        return outs[0], outs[1]
    return outs[0], None


def _dense_body(acc, cnt, x, wl, wr, b, o):
    dh = acc.shape[2]
    inv = 1.0 / jnp.maximum(cnt[:, 0:1], 1.0)
    y = (jnp.dot(acc[0] * inv, wl[0:dh], preferred_element_type=jnp.float32,
                 precision=_PREC)
         + jnp.dot(acc[1] * inv, wl[dh:2 * dh],
                   preferred_element_type=jnp.float32, precision=_PREC)
         + jnp.dot(x[...], wr[...], preferred_element_type=jnp.float32,
                   precision=_PREC)
         + b[...])
    h = jnp.maximum(y, 0.0)
    o[0] = h[:, 0:dh]
    o[1] = h[:, dh:2 * dh]


def _tc_dense(acc, cnt, x, wl, wr, b, br=2000):
    n, d = x.shape
    h = wl.shape[1]
    dh = d // 2
    grid = (n // br,)
    row3 = lambda i: (0, i, 0)
    row = lambda i: (i, 0)
    full = lambda i: (0, 0)
    return pl.pallas_call(
        _dense_body,
        grid=grid,
        in_specs=[
            pl.BlockSpec((2, br, dh), row3),
            pl.BlockSpec((br, CW), row),
            pl.BlockSpec((br, d), row),
            pl.BlockSpec((d, h), full),
            pl.BlockSpec((d, h), full),
            pl.BlockSpec((1, h), full),
        ],
        out_specs=pl.BlockSpec((2, br, h // 2), row3),
        out_shape=jax.ShapeDtypeStruct((2, n, h // 2), jnp.float32),
    )(acc, cnt, x, wl, wr, b)


def _final_body(acc, cnt, hin, wl, wr, b, batch, bnw, bnb, fcw, fcb,
                o, pooled):
    i = pl.program_id(0)
    nsteps = pl.num_programs(0)
    dh = acc.shape[2]
    inv = 1.0 / jnp.maximum(cnt[:, 0:1], 1.0)
    y = (jnp.dot(acc[0] * inv, wl[0:dh], preferred_element_type=jnp.float32,
                 precision=_PREC)
         + jnp.dot(acc[1] * inv, wl[dh:2 * dh],
                   preferred_element_type=jnp.float32, precision=_PREC)
         + jnp.dot(hin[0], wr[0:dh], preferred_element_type=jnp.float32,
                   precision=_PREC)
         + jnp.dot(hin[1], wr[dh:2 * dh], preferred_element_type=jnp.float32,
                   precision=_PREC)
         + b[...])
    h2 = jnp.maximum(y, 0.0)
    br = h2.shape[0]
    gid = lax.broadcasted_iota(jnp.int32, (G, br), 0).astype(jnp.float32)
    sel = (batch[...].reshape(1, br) == gid).astype(jnp.float32)
    part = jnp.dot(sel, h2, preferred_element_type=jnp.float32,
                   precision=_PREC)

    @pl.when(i == 0)
    def _():
        pooled[...] = part

    @pl.when(i > 0)
    def _():
        pooled[...] += part

    @pl.when(i == nsteps - 1)
    def _():
        pool = pooled[...] * (1.0 / (1.0 + EPS) ** 0.5) * bnw[...] + bnb[...]
        o[...] = (jnp.dot(pool, fcw[...], preferred_element_type=jnp.float32,
                          precision=_PREC) + fcb[...])


def _tc_final(acc, cnt, hin, wl, wr, b, batch_f, bnw, bnb, fcw, fcb, br=2000):
    _, n, dh = hin.shape
    h = wl.shape[1]
    l = fcw.shape[1]
    grid = (n // br,)
    row3 = lambda i: (0, i, 0)
    row = lambda i: (i, 0)
    full = lambda i: (0, 0)
    return pl.pallas_call(
        _final_body,
        grid=grid,
        in_specs=[
            pl.BlockSpec((2, br, dh), row3),
            pl.BlockSpec((br, CW), row),
            pl.BlockSpec((2, br, dh), row3),
            pl.BlockSpec((h, h), full),
            pl.BlockSpec((h, h), full),
            pl.BlockSpec((1, h), full),
            pl.BlockSpec((1, 1, br), lambda i: (i, 0, 0)),
            pl.BlockSpec((1, h), full),
            pl.BlockSpec((1, h), full),
            pl.BlockSpec((h, l), full),
            pl.BlockSpec((1, l), full),
        ],
        out_specs=pl.BlockSpec((G, l), full),
        out_shape=jax.ShapeDtypeStruct((G, l), jnp.float32),
        scratch_shapes=[pltpu.VMEM((G, h), jnp.float32)],
    )(acc, cnt, hin, wl, wr, b, batch_f, bnw, bnb, fcw, fcb)


def kernel(x, edge_index, batch, W_l1, W_r1, b1, W_l2, W_r2, b2,
           bn_w, bn_b, fc_w, fc_b):
    n, d = x.shape
    dh = d // 2
    src = edge_index[0]
    dst = edge_index[1]
    ones = jnp.ones((CHUNK, CW), jnp.float32)
    rpt = (n // NS) // 8 * 8
    zf = jnp.zeros((rpt, dh), jnp.float32)
    zcf = jnp.zeros((rpt, CW), jnp.float32)

    x2 = jnp.stack([x[:, :dh], x[:, dh:]]).reshape(NC * n, dh)
    acc1, cnt = _sc_agg(x2, src, dst, ones, zf, zcf, with_counts=True)
    h1 = _tc_dense(acc1.reshape(NC, n, dh), cnt, x,
                   W_l1, W_r1, b1.reshape(1, -1))
    acc2, _ = _sc_agg(h1.reshape(NC * n, dh), src, dst, ones, zf, zcf,
                      with_counts=False)
    batch_f = batch.astype(jnp.float32).reshape(n // 2000, 1, 2000)
    out = _tc_final(acc2.reshape(NC, n, dh), cnt, h1,
                    W_l2, W_r2, b2.reshape(1, -1), batch_f,
                    bn_w.reshape(1, -1), bn_b.reshape(1, -1),
                    fc_w, fc_b.reshape(1, -1))
    return out
